# baseline (speedup 1.0000x reference)
import jax
import jax.numpy as jnp
from jax import lax
from jax.experimental import pallas as pl
from jax.experimental.pallas import tpu as pltpu
from jax.experimental.pallas import tpu_sc as plsc

N = 10000
E = 320000
D = 128

NC = 2
NS = 16
NW = NC * NS

EPC = 128
NCH = E // EPC
CPW = NCH // NW
NTAIL = NCH - CPW * NW
NP_ = 10240
RPS = NP_ // NS

_mesh = plsc.VectorSubcoreMesh(core_axis_name="c", subcore_axis_name="s")


def _sc_aggregate(x, ef, zrows):
    def body(x_hbm, ef_hbm, z_hbm, acc_out,
             sa0, sa1, da0, da1, rows0, rows1, acc_sh, sem):
        c = lax.axis_index("c")
        s = lax.axis_index("s")
        wid = c * NS + s

        pltpu.sync_copy(z_hbm, acc_sh.at[pl.ds(s * RPS, RPS)])
        plsc.subcore_barrier()

        o0 = wid * EPC
        pltpu.sync_copy(ef_hbm.at[pl.ds(o0, EPC)], sa0)
        pltpu.sync_copy(ef_hbm.at[pl.ds(E + o0, EPC)], da0)
        pltpu.async_copy(x_hbm.at[sa0], rows0, sem)

        @pl.loop(0, CPW, step=2)
        def _(ci):
            o1 = (wid + NW * (ci + 1)) * EPC
            pltpu.sync_copy(ef_hbm.at[pl.ds(o1, EPC)], sa1)
            pltpu.sync_copy(ef_hbm.at[pl.ds(E + o1, EPC)], da1)
            pltpu.make_async_copy(x_hbm.at[sa0], rows0, sem).wait()
            pltpu.async_copy(x_hbm.at[sa1], rows1, sem)
            pltpu.sync_copy(rows0, acc_sh.at[da0], add=True)

            @pl.when(ci + 2 < CPW)
            def _():
                o2 = (wid + NW * (ci + 2)) * EPC
                pltpu.sync_copy(ef_hbm.at[pl.ds(o2, EPC)], sa0)
                pltpu.sync_copy(ef_hbm.at[pl.ds(E + o2, EPC)], da0)

            pltpu.make_async_copy(x_hbm.at[sa1], rows1, sem).wait()

            @pl.when(ci + 2 < CPW)
            def _():
                pltpu.async_copy(x_hbm.at[sa0], rows0, sem)

            pltpu.sync_copy(rows1, acc_sh.at[da1], add=True)

        @pl.when(wid < NTAIL)
        def _():
            ot = (NCH - NTAIL + wid) * EPC
            pltpu.sync_copy(ef_hbm.at[pl.ds(ot, EPC)], sa0)
            pltpu.sync_copy(ef_hbm.at[pl.ds(E + ot, EPC)], da0)
            pltpu.async_copy(x_hbm.at[sa0], rows0, sem).wait()
            pltpu.sync_copy(rows0, acc_sh.at[da0], add=True)

        plsc.subcore_barrier()
        pltpu.sync_copy(acc_sh.at[pl.ds(s * RPS, RPS)],
                        acc_out.at[pl.ds(c * NP_ + s * RPS, RPS)])

    run = pl.kernel(body, out_type=jax.ShapeDtypeStruct((NC * NP_, D), jnp.float32),
                    mesh=_mesh,
                    scratch_types=[
                        pltpu.VMEM((EPC,), jnp.int32),
                        pltpu.VMEM((EPC,), jnp.int32),
                        pltpu.VMEM((EPC,), jnp.int32),
                        pltpu.VMEM((EPC,), jnp.int32),
                        pltpu.VMEM((EPC, D), jnp.float32),
                        pltpu.VMEM((EPC, D), jnp.float32),
                        pltpu.VMEM_SHARED((NP_, D), jnp.float32),
                        pltpu.SemaphoreType.DMA,
                    ])
    return run(x, ef, zrows)


def _sc_degree(ef, zrows, ones_rows):
    def body(ef_hbm, z_hbm, ones_hbm, deg_out,
             didx0, didx1, didx2, didx3, ones_v, acc_sh, sem):
        c = lax.axis_index("c")
        s = lax.axis_index("s")
        wid = c * NS + s

        pltpu.sync_copy(z_hbm, acc_sh.at[pl.ds(s * RPS, RPS)])
        pltpu.sync_copy(ones_hbm, ones_v)
        plsc.subcore_barrier()

        dbufs = (didx0, didx1, didx2, didx3)
        for k in range(4):
            pltpu.sync_copy(ef_hbm.at[pl.ds(E + (wid + NW * k) * EPC, EPC)],
                            dbufs[k])

        @pl.loop(0, 76, step=4)
        def _(ci):
            descs = [pltpu.async_copy(ones_v, acc_sh.at[dbufs[k]], sem,
                                      add=True) for k in range(4)]
            for k in range(4):
                descs[k].wait()

                @pl.when(ci + 4 + k < CPW)
                def _(k=k):
                    o = E + (wid + NW * (ci + 4 + k)) * EPC
                    pltpu.sync_copy(ef_hbm.at[pl.ds(o, EPC)], dbufs[k])

        for k in range(2):
            pltpu.sync_copy(ones_v, acc_sh.at[dbufs[k]], add=True)

        @pl.when(wid < NTAIL)
        def _():
            ot = E + (NCH - NTAIL + wid) * EPC
            pltpu.sync_copy(ef_hbm.at[pl.ds(ot, EPC)], didx0)
            pltpu.sync_copy(ones_v, acc_sh.at[didx0], add=True)

        plsc.subcore_barrier()
        pltpu.sync_copy(acc_sh.at[pl.ds(s * RPS, RPS)],
                        deg_out.at[pl.ds(c * NP_ + s * RPS, RPS)])

    run = pl.kernel(body, out_type=jax.ShapeDtypeStruct((NC * NP_, D), jnp.float32),
                    mesh=_mesh,
                    scratch_types=[
                        pltpu.VMEM((EPC,), jnp.int32),
                        pltpu.VMEM((EPC,), jnp.int32),
                        pltpu.VMEM((EPC,), jnp.int32),
                        pltpu.VMEM((EPC,), jnp.int32),
                        pltpu.VMEM((EPC, D), jnp.float32),
                        pltpu.VMEM_SHARED((NP_, D), jnp.float32),
                        pltpu.SemaphoreType.DMA,
                    ])
    return run(ef, zrows, ones_rows)


_BR = 1000


def _tc_root(x, W_r, b):
    def body(x_ref, wr_ref, b_ref, o_ref):
        o_ref[...] = jnp.dot(x_ref[...], wr_ref[...],
                             preferred_element_type=jnp.float32) + b_ref[...]

    return pl.pallas_call(
        body,
        grid=(N // _BR,),
        in_specs=[
            pl.BlockSpec((_BR, D), lambda i: (i, 0)),
            pl.BlockSpec((D, D), lambda i: (0, 0)),
            pl.BlockSpec((1, D), lambda i: (0, 0)),
        ],
        out_specs=pl.BlockSpec((_BR, D), lambda i: (i, 0)),
        out_shape=jax.ShapeDtypeStruct((N, D), jnp.float32),
    )(x, W_r, b)


def _tc_combine(acc, degw, xr, W_l, relu):
    def body(acc_ref, deg_ref, xr_ref, wl_ref, o_ref):
        s = acc_ref[0] + acc_ref[1]
        deg = deg_ref[0, :, 0] + deg_ref[1, :, 0]
        mean = s / jnp.clip(deg, 1.0, None)[:, None]
        y = (jnp.dot(mean, wl_ref[...], preferred_element_type=jnp.float32)
             + xr_ref[...])
        o_ref[...] = jnp.maximum(y, 0.0) if relu else y

    return pl.pallas_call(
        body,
        grid=(N // _BR,),
        in_specs=[
            pl.BlockSpec((NC, _BR, D), lambda i: (0, i, 0)),
            pl.BlockSpec((NC, _BR, D), lambda i: (0, i, 0)),
            pl.BlockSpec((_BR, D), lambda i: (i, 0)),
            pl.BlockSpec((D, D), lambda i: (0, 0)),
        ],
        out_specs=pl.BlockSpec((_BR, D), lambda i: (i, 0)),
        out_shape=jax.ShapeDtypeStruct((N, D), jnp.float32),
    )(acc, degw, xr, W_l)


@jax.jit
def kernel(x, edge_index, W_l1, W_r1, b1, W_l2, W_r2, b2):
    ef = edge_index.astype(jnp.int32).reshape(2 * E)
    zrows = jnp.zeros((RPS, D), jnp.float32)
    ones_rows = jnp.ones((EPC, D), jnp.float32)
    b1r = b1.reshape(1, D)
    b2r = b2.reshape(1, D)

    degw = _sc_degree(ef, zrows, ones_rows).reshape(NC, NP_, D)
    xr1 = _tc_root(x, W_r1, b1r)
    acc1 = _sc_aggregate(x, ef, zrows).reshape(NC, NP_, D)
    h = _tc_combine(acc1, degw, xr1, W_l1, True)
    xr2 = _tc_root(h, W_r2, b2r)
    acc2 = _sc_aggregate(h, ef, zrows).reshape(NC, NP_, D)
    out = _tc_combine(acc2, degw, xr2, W_l2, False)
    return out


# speedup vs baseline: 1.0161x; 1.0161x over previous
import jax
import jax.numpy as jnp
from jax import lax
from jax.experimental import pallas as pl
from jax.experimental.pallas import tpu as pltpu
from jax.experimental.pallas import tpu_sc as plsc

N = 10000
E = 320000
D = 128

NC = 2
NS = 16
NW = NC * NS

EPC = 128
NCH = E // EPC
CPW = NCH // NW
NTAIL = NCH - CPW * NW
NP_ = 10240
RPS = NP_ // NS

_mesh = plsc.VectorSubcoreMesh(core_axis_name="c", subcore_axis_name="s")


def _sc_aggregate(x, ef, zrows):
    def body(x_hbm, ef_hbm, z_hbm, acc_out,
             sa0, sa1, da0, da1, rows0, rows1, acc_sh, sem):
        c = lax.axis_index("c")
        s = lax.axis_index("s")
        wid = c * NS + s

        pltpu.sync_copy(z_hbm, acc_sh.at[pl.ds(s * RPS, RPS)])
        plsc.subcore_barrier()

        o0 = wid * EPC
        pltpu.sync_copy(ef_hbm.at[pl.ds(o0, EPC)], sa0)
        pltpu.sync_copy(ef_hbm.at[pl.ds(E + o0, EPC)], da0)
        pltpu.async_copy(x_hbm.at[sa0], rows0, sem)

        @pl.loop(0, CPW, step=2)
        def _(ci):
            o1 = (wid + NW * (ci + 1)) * EPC
            pltpu.sync_copy(ef_hbm.at[pl.ds(o1, EPC)], sa1)
            pltpu.sync_copy(ef_hbm.at[pl.ds(E + o1, EPC)], da1)
            pltpu.make_async_copy(x_hbm.at[sa0], rows0, sem).wait()
            pltpu.async_copy(x_hbm.at[sa1], rows1, sem)
            pltpu.sync_copy(rows0, acc_sh.at[da0], add=True)

            @pl.when(ci + 2 < CPW)
            def _():
                o2 = (wid + NW * (ci + 2)) * EPC
                pltpu.sync_copy(ef_hbm.at[pl.ds(o2, EPC)], sa0)
                pltpu.sync_copy(ef_hbm.at[pl.ds(E + o2, EPC)], da0)

            pltpu.make_async_copy(x_hbm.at[sa1], rows1, sem).wait()

            @pl.when(ci + 2 < CPW)
            def _():
                pltpu.async_copy(x_hbm.at[sa0], rows0, sem)

            pltpu.sync_copy(rows1, acc_sh.at[da1], add=True)

        @pl.when(wid < NTAIL)
        def _():
            ot = (NCH - NTAIL + wid) * EPC
            pltpu.sync_copy(ef_hbm.at[pl.ds(ot, EPC)], sa0)
            pltpu.sync_copy(ef_hbm.at[pl.ds(E + ot, EPC)], da0)
            pltpu.async_copy(x_hbm.at[sa0], rows0, sem).wait()
            pltpu.sync_copy(rows0, acc_sh.at[da0], add=True)

        plsc.subcore_barrier()
        pltpu.sync_copy(acc_sh.at[pl.ds(s * RPS, RPS)],
                        acc_out.at[pl.ds(c * NP_ + s * RPS, RPS)])

    run = pl.kernel(body, out_type=jax.ShapeDtypeStruct((NC * NP_, D), jnp.float32),
                    mesh=_mesh,
                    scratch_types=[
                        pltpu.VMEM((EPC,), jnp.int32),
                        pltpu.VMEM((EPC,), jnp.int32),
                        pltpu.VMEM((EPC,), jnp.int32),
                        pltpu.VMEM((EPC,), jnp.int32),
                        pltpu.VMEM((EPC, D), jnp.float32),
                        pltpu.VMEM((EPC, D), jnp.float32),
                        pltpu.VMEM_SHARED((NP_, D), jnp.float32),
                        pltpu.SemaphoreType.DMA,
                    ])
    return run(x, ef, zrows)


def _sc_degree(ef, zrows, ones_rows):
    def body(ef_hbm, z_hbm, ones_hbm, deg_out, didx0, didx1, ones_v, acc_sh, sem):
        c = lax.axis_index("c")
        s = lax.axis_index("s")
        wid = c * NS + s

        pltpu.sync_copy(z_hbm, acc_sh.at[pl.ds(s * RPS, RPS)])
        pltpu.sync_copy(ones_hbm, ones_v)
        plsc.subcore_barrier()

        pltpu.sync_copy(ef_hbm.at[pl.ds(E + wid * EPC, EPC)], didx0)

        @pl.loop(0, CPW, step=2)
        def _(ci):
            d0 = pltpu.async_copy(ones_v, acc_sh.at[didx0], sem, add=True)
            o1 = E + (wid + NW * (ci + 1)) * EPC
            pltpu.sync_copy(ef_hbm.at[pl.ds(o1, EPC)], didx1)
            d0.wait()
            d1 = pltpu.async_copy(ones_v, acc_sh.at[didx1], sem, add=True)

            @pl.when(ci + 2 < CPW)
            def _():
                o2 = E + (wid + NW * (ci + 2)) * EPC
                pltpu.sync_copy(ef_hbm.at[pl.ds(o2, EPC)], didx0)

            d1.wait()

        @pl.when(wid < NTAIL)
        def _():
            ot = E + (NCH - NTAIL + wid) * EPC
            pltpu.sync_copy(ef_hbm.at[pl.ds(ot, EPC)], didx0)
            pltpu.sync_copy(ones_v, acc_sh.at[didx0], add=True)

        plsc.subcore_barrier()
        pltpu.sync_copy(acc_sh.at[pl.ds(s * RPS, RPS)],
                        deg_out.at[pl.ds(c * NP_ + s * RPS, RPS)])

    run = pl.kernel(body, out_type=jax.ShapeDtypeStruct((NC * NP_, D), jnp.float32),
                    mesh=_mesh,
                    scratch_types=[
                        pltpu.VMEM((EPC,), jnp.int32),
                        pltpu.VMEM((EPC,), jnp.int32),
                        pltpu.VMEM((EPC, D), jnp.float32),
                        pltpu.VMEM_SHARED((NP_, D), jnp.float32),
                        pltpu.SemaphoreType.DMA,
                    ])
    return run(ef, zrows, ones_rows)


_BR = 1000


def _tc_root(x, W_r, b):
    def body(x_ref, wr_ref, b_ref, o_ref):
        o_ref[...] = jnp.dot(x_ref[...], wr_ref[...],
                             preferred_element_type=jnp.float32) + b_ref[...]

    return pl.pallas_call(
        body,
        grid=(N // _BR,),
        in_specs=[
            pl.BlockSpec((_BR, D), lambda i: (i, 0)),
            pl.BlockSpec((D, D), lambda i: (0, 0)),
            pl.BlockSpec((1, D), lambda i: (0, 0)),
        ],
        out_specs=pl.BlockSpec((_BR, D), lambda i: (i, 0)),
        out_shape=jax.ShapeDtypeStruct((N, D), jnp.float32),
    )(x, W_r, b)


def _tc_combine(acc, degw, xr, W_l, relu):
    def body(acc_ref, deg_ref, xr_ref, wl_ref, o_ref):
        s = acc_ref[0] + acc_ref[1]
        deg = deg_ref[0, :, 0] + deg_ref[1, :, 0]
        mean = s / jnp.clip(deg, 1.0, None)[:, None]
        y = (jnp.dot(mean, wl_ref[...], preferred_element_type=jnp.float32)
             + xr_ref[...])
        o_ref[...] = jnp.maximum(y, 0.0) if relu else y

    return pl.pallas_call(
        body,
        grid=(N // _BR,),
        in_specs=[
            pl.BlockSpec((NC, _BR, D), lambda i: (0, i, 0)),
            pl.BlockSpec((NC, _BR, D), lambda i: (0, i, 0)),
            pl.BlockSpec((_BR, D), lambda i: (i, 0)),
            pl.BlockSpec((D, D), lambda i: (0, 0)),
        ],
        out_specs=pl.BlockSpec((_BR, D), lambda i: (i, 0)),
        out_shape=jax.ShapeDtypeStruct((N, D), jnp.float32),
    )(acc, degw, xr, W_l)


@jax.jit
def kernel(x, edge_index, W_l1, W_r1, b1, W_l2, W_r2, b2):
    ef = edge_index.astype(jnp.int32).reshape(2 * E)
    zrows = jnp.zeros((RPS, D), jnp.float32)
    ones_rows = jnp.ones((EPC, D), jnp.float32)
    b1r = b1.reshape(1, D)
    b2r = b2.reshape(1, D)

    degw = _sc_degree(ef, zrows, ones_rows).reshape(NC, NP_, D)
    xr1 = _tc_root(x, W_r1, b1r)
    acc1 = _sc_aggregate(x, ef, zrows).reshape(NC, NP_, D)
    h = _tc_combine(acc1, degw, xr1, W_l1, True)
    xr2 = _tc_root(h, W_r2, b2r)
    acc2 = _sc_aggregate(h, ef, zrows).reshape(NC, NP_, D)
    out = _tc_combine(acc2, degw, xr2, W_l2, False)
    return out


# speedup vs baseline: 1.0434x; 1.0268x over previous
import jax
import jax.numpy as jnp
from jax import lax
from jax.experimental import pallas as pl
from jax.experimental.pallas import tpu as pltpu
from jax.experimental.pallas import tpu_sc as plsc

N = 10000
E = 320000
D = 128

NC = 2
NS = 16
NW = NC * NS

EPC = 128
NCH = E // EPC
CPW = NCH // NW
NTAIL = NCH - CPW * NW
NP_ = 10240
RPS = NP_ // NS

_mesh = plsc.VectorSubcoreMesh(core_axis_name="c", subcore_axis_name="s")


def _sc_aggregate(x, ef, zrows):
    def body(x_hbm, ef_hbm, z_hbm, acc_out,
             sa0, sa1, da0, da1, rows0, rows1, acc_sh, sem):
        c = lax.axis_index("c")
        s = lax.axis_index("s")
        wid = c * NS + s

        pltpu.sync_copy(z_hbm, acc_sh.at[pl.ds(s * RPS, RPS)])
        plsc.subcore_barrier()

        o0 = wid * EPC
        pltpu.sync_copy(ef_hbm.at[pl.ds(o0, EPC)], sa0)
        pltpu.sync_copy(ef_hbm.at[pl.ds(E + o0, EPC)], da0)
        pltpu.async_copy(x_hbm.at[sa0], rows0, sem)

        @pl.loop(0, CPW, step=2)
        def _(ci):
            o1 = (wid + NW * (ci + 1)) * EPC
            pltpu.sync_copy(ef_hbm.at[pl.ds(o1, EPC)], sa1)
            pltpu.sync_copy(ef_hbm.at[pl.ds(E + o1, EPC)], da1)
            pltpu.make_async_copy(x_hbm.at[sa0], rows0, sem).wait()
            pltpu.async_copy(x_hbm.at[sa1], rows1, sem)
            pltpu.sync_copy(rows0, acc_sh.at[da0], add=True)

            @pl.when(ci + 2 < CPW)
            def _():
                o2 = (wid + NW * (ci + 2)) * EPC
                pltpu.sync_copy(ef_hbm.at[pl.ds(o2, EPC)], sa0)
                pltpu.sync_copy(ef_hbm.at[pl.ds(E + o2, EPC)], da0)

            pltpu.make_async_copy(x_hbm.at[sa1], rows1, sem).wait()

            @pl.when(ci + 2 < CPW)
            def _():
                pltpu.async_copy(x_hbm.at[sa0], rows0, sem)

            pltpu.sync_copy(rows1, acc_sh.at[da1], add=True)

        @pl.when(wid < NTAIL)
        def _():
            ot = (NCH - NTAIL + wid) * EPC
            pltpu.sync_copy(ef_hbm.at[pl.ds(ot, EPC)], sa0)
            pltpu.sync_copy(ef_hbm.at[pl.ds(E + ot, EPC)], da0)
            pltpu.async_copy(x_hbm.at[sa0], rows0, sem).wait()
            pltpu.sync_copy(rows0, acc_sh.at[da0], add=True)

        plsc.subcore_barrier()
        pltpu.sync_copy(acc_sh.at[pl.ds(s * RPS, RPS)],
                        acc_out.at[pl.ds(c * NP_ + s * RPS, RPS)])

    run = pl.kernel(body, out_type=jax.ShapeDtypeStruct((NC * NP_, D), jnp.float32),
                    mesh=_mesh,
                    scratch_types=[
                        pltpu.VMEM((EPC,), jnp.int32),
                        pltpu.VMEM((EPC,), jnp.int32),
                        pltpu.VMEM((EPC,), jnp.int32),
                        pltpu.VMEM((EPC,), jnp.int32),
                        pltpu.VMEM((EPC, D), jnp.float32),
                        pltpu.VMEM((EPC, D), jnp.float32),
                        pltpu.VMEM_SHARED((NP_, D), jnp.float32),
                        pltpu.SemaphoreType.DMA,
                    ])
    return run(x, ef, zrows)


def _sc_degree(ef, zrows, ones_rows):
    def body(ef_hbm, z_hbm, ones_hbm, deg_out, didx0, didx1, ones_v, acc_sh, sem):
        c = lax.axis_index("c")
        s = lax.axis_index("s")
        wid = c * NS + s

        pltpu.sync_copy(z_hbm, acc_sh.at[pl.ds(s * RPS, RPS)])
        pltpu.sync_copy(ones_hbm, ones_v)
        plsc.subcore_barrier()

        pltpu.sync_copy(ef_hbm.at[pl.ds(E + wid * EPC, EPC)], didx0)

        @pl.loop(0, CPW, step=2)
        def _(ci):
            d0 = pltpu.async_copy(ones_v, acc_sh.at[didx0], sem, add=True)
            o1 = E + (wid + NW * (ci + 1)) * EPC
            pltpu.sync_copy(ef_hbm.at[pl.ds(o1, EPC)], didx1)
            d0.wait()
            d1 = pltpu.async_copy(ones_v, acc_sh.at[didx1], sem, add=True)

            @pl.when(ci + 2 < CPW)
            def _():
                o2 = E + (wid + NW * (ci + 2)) * EPC
                pltpu.sync_copy(ef_hbm.at[pl.ds(o2, EPC)], didx0)

            d1.wait()

        @pl.when(wid < NTAIL)
        def _():
            ot = E + (NCH - NTAIL + wid) * EPC
            pltpu.sync_copy(ef_hbm.at[pl.ds(ot, EPC)], didx0)
            pltpu.sync_copy(ones_v, acc_sh.at[didx0], add=True)

        plsc.subcore_barrier()
        pltpu.sync_copy(acc_sh.at[pl.ds(s * RPS, RPS)],
                        deg_out.at[pl.ds(c * NP_ + s * RPS, RPS)])

    run = pl.kernel(body, out_type=jax.ShapeDtypeStruct((NC * NP_, D), jnp.float32),
                    mesh=_mesh,
                    scratch_types=[
                        pltpu.VMEM((EPC,), jnp.int32),
                        pltpu.VMEM((EPC,), jnp.int32),
                        pltpu.VMEM((EPC, D), jnp.float32),
                        pltpu.VMEM_SHARED((NP_, D), jnp.float32),
                        pltpu.SemaphoreType.DMA,
                    ])
    return run(ef, zrows, ones_rows)


_BR = 1000


def _tc_combine(acc, degw, x, W_l, W_r, b, relu):
    def body(acc_ref, deg_ref, x_ref, wl_ref, wr_ref, b_ref, o_ref):
        s = acc_ref[0] + acc_ref[1]
        deg = deg_ref[0, :, 0] + deg_ref[1, :, 0]
        mean = s / jnp.clip(deg, 1.0, None)[:, None]
        y = (jnp.dot(mean, wl_ref[...], preferred_element_type=jnp.float32)
             + jnp.dot(x_ref[...], wr_ref[...], preferred_element_type=jnp.float32)
             + b_ref[...])
        o_ref[...] = jnp.maximum(y, 0.0) if relu else y

    return pl.pallas_call(
        body,
        grid=(N // _BR,),
        in_specs=[
            pl.BlockSpec((NC, _BR, D), lambda i: (0, i, 0)),
            pl.BlockSpec((NC, _BR, D), lambda i: (0, i, 0)),
            pl.BlockSpec((_BR, D), lambda i: (i, 0)),
            pl.BlockSpec((D, D), lambda i: (0, 0)),
            pl.BlockSpec((D, D), lambda i: (0, 0)),
            pl.BlockSpec((1, D), lambda i: (0, 0)),
        ],
        out_specs=pl.BlockSpec((_BR, D), lambda i: (i, 0)),
        out_shape=jax.ShapeDtypeStruct((N, D), jnp.float32),
    )(acc, degw, x, W_l, W_r, b)


@jax.jit
def kernel(x, edge_index, W_l1, W_r1, b1, W_l2, W_r2, b2):
    ef = edge_index.astype(jnp.int32).reshape(2 * E)
    zrows = jnp.zeros((RPS, D), jnp.float32)
    ones_rows = jnp.ones((EPC, D), jnp.float32)
    b1r = b1.reshape(1, D)
    b2r = b2.reshape(1, D)

    degw = _sc_degree(ef, zrows, ones_rows).reshape(NC, NP_, D)
    acc1 = _sc_aggregate(x, ef, zrows).reshape(NC, NP_, D)
    h = _tc_combine(acc1, degw, x, W_l1, W_r1, b1r, True)
    acc2 = _sc_aggregate(h, ef, zrows).reshape(NC, NP_, D)
    out = _tc_combine(acc2, degw, h, W_l2, W_r2, b2r, False)
    return out


# speedup vs baseline: 1.1198x; 1.0733x over previous
import jax
import jax.numpy as jnp
from jax import lax
from jax.experimental import pallas as pl
from jax.experimental.pallas import tpu as pltpu
from jax.experimental.pallas import tpu_sc as plsc

N = 10000
E = 320000
D = 128

NC = 2
NS = 16
NW = NC * NS

EPC = 100
NCH = E // EPC
CPW = NCH // NW
NP_ = 10240
RPS = NP_ // NS

_mesh = plsc.VectorSubcoreMesh(core_axis_name="c", subcore_axis_name="s")


def _sc_aggregate(x, eint, zrows):
    def body(x_hbm, e_hbm, z_hbm, acc_out,
             ib0, ib1, ib2, rows0, rows1, rows2, acc_sh, sg0, sg1, sg2):
        c = lax.axis_index("c")
        s = lax.axis_index("s")
        wid = c * NS + s

        pltpu.sync_copy(z_hbm, acc_sh.at[pl.ds(s * RPS, RPS)])
        plsc.subcore_barrier()

        ibs = (ib0, ib1, ib2)
        rbs = (rows0, rows1, rows2)
        sgs = (sg0, sg1, sg2)

        pltpu.sync_copy(e_hbm.at[wid], ib0)
        pltpu.async_copy(x_hbm.at[ib0.at[0]], rows0, sg0)
        pltpu.sync_copy(e_hbm.at[wid + NW], ib1)
        pltpu.async_copy(x_hbm.at[ib1.at[0]], rows1, sg1)

        @pl.loop(0, CPW, step=3)
        def _(ci):
            for k in range(3):
                kk = (k + 2) % 3

                @pl.when(ci + k + 2 < CPW)
                def _(k=k, kk=kk):
                    pltpu.sync_copy(e_hbm.at[wid + NW * (ci + k + 2)], ibs[kk])
                    pltpu.async_copy(x_hbm.at[ibs[kk].at[0]], rbs[kk], sgs[kk])

                @pl.when(ci + k < CPW)
                def _(k=k):
                    pltpu.make_async_copy(x_hbm.at[ibs[k].at[0]], rbs[k],
                                          sgs[k]).wait()
                    pltpu.sync_copy(rbs[k], acc_sh.at[ibs[k].at[1]], add=True)

        plsc.subcore_barrier()
        pltpu.sync_copy(acc_sh.at[pl.ds(s * RPS, RPS)],
                        acc_out.at[pl.ds(c * NP_ + s * RPS, RPS)])

    run = pl.kernel(body, out_type=jax.ShapeDtypeStruct((NC * NP_, D), jnp.float32),
                    mesh=_mesh,
                    scratch_types=[
                        pltpu.VMEM((2, EPC), jnp.int32),
                        pltpu.VMEM((2, EPC), jnp.int32),
                        pltpu.VMEM((2, EPC), jnp.int32),
                        pltpu.VMEM((EPC, D), jnp.float32),
                        pltpu.VMEM((EPC, D), jnp.float32),
                        pltpu.VMEM((EPC, D), jnp.float32),
                        pltpu.VMEM_SHARED((NP_, D), jnp.float32),
                        pltpu.SemaphoreType.DMA,
                        pltpu.SemaphoreType.DMA,
                        pltpu.SemaphoreType.DMA,
                    ])
    return run(x, eint, zrows)


def _sc_degree(eint, zrows, ones_rows):
    def body(e_hbm, z_hbm, ones_hbm, deg_out, ib0, ib1, ones_v, acc_sh, sem):
        c = lax.axis_index("c")
        s = lax.axis_index("s")
        wid = c * NS + s

        pltpu.sync_copy(z_hbm, acc_sh.at[pl.ds(s * RPS, RPS)])
        pltpu.sync_copy(ones_hbm, ones_v)
        plsc.subcore_barrier()

        pltpu.sync_copy(e_hbm.at[wid], ib0)

        @pl.loop(0, CPW, step=2)
        def _(ci):
            d0 = pltpu.async_copy(ones_v, acc_sh.at[ib0.at[1]], sem, add=True)
            pltpu.sync_copy(e_hbm.at[wid + NW * (ci + 1)], ib1)
            d0.wait()
            d1 = pltpu.async_copy(ones_v, acc_sh.at[ib1.at[1]], sem, add=True)

            @pl.when(ci + 2 < CPW)
            def _():
                pltpu.sync_copy(e_hbm.at[wid + NW * (ci + 2)], ib0)

            d1.wait()

        plsc.subcore_barrier()
        pltpu.sync_copy(acc_sh.at[pl.ds(s * RPS, RPS)],
                        deg_out.at[pl.ds(c * NP_ + s * RPS, RPS)])

    run = pl.kernel(body, out_type=jax.ShapeDtypeStruct((NC * NP_, D), jnp.float32),
                    mesh=_mesh,
                    scratch_types=[
                        pltpu.VMEM((2, EPC), jnp.int32),
                        pltpu.VMEM((2, EPC), jnp.int32),
                        pltpu.VMEM((EPC, D), jnp.float32),
                        pltpu.VMEM_SHARED((NP_, D), jnp.float32),
                        pltpu.SemaphoreType.DMA,
                    ])
    return run(eint, zrows, ones_rows)


_BR = 1000


def _tc_combine(acc, degw, x, W_l, W_r, b, relu):
    def body(acc_ref, deg_ref, x_ref, wl_ref, wr_ref, b_ref, o_ref):
        s = acc_ref[0] + acc_ref[1]
        deg = deg_ref[0, :, 0] + deg_ref[1, :, 0]
        mean = s / jnp.clip(deg, 1.0, None)[:, None]
        y = (jnp.dot(mean, wl_ref[...], preferred_element_type=jnp.float32)
             + jnp.dot(x_ref[...], wr_ref[...], preferred_element_type=jnp.float32)
             + b_ref[...])
        o_ref[...] = jnp.maximum(y, 0.0) if relu else y

    return pl.pallas_call(
        body,
        grid=(N // _BR,),
        in_specs=[
            pl.BlockSpec((NC, _BR, D), lambda i: (0, i, 0)),
            pl.BlockSpec((NC, _BR, D), lambda i: (0, i, 0)),
            pl.BlockSpec((_BR, D), lambda i: (i, 0)),
            pl.BlockSpec((D, D), lambda i: (0, 0)),
            pl.BlockSpec((D, D), lambda i: (0, 0)),
            pl.BlockSpec((1, D), lambda i: (0, 0)),
        ],
        out_specs=pl.BlockSpec((_BR, D), lambda i: (i, 0)),
        out_shape=jax.ShapeDtypeStruct((N, D), jnp.float32),
    )(acc, degw, x, W_l, W_r, b)


@jax.jit
def kernel(x, edge_index, W_l1, W_r1, b1, W_l2, W_r2, b2):
    ei = edge_index.astype(jnp.int32)
    zrows = jnp.zeros((RPS, D), jnp.float32)
    ones_rows = jnp.ones((EPC, D), jnp.float32)
    b1r = b1.reshape(1, D)
    b2r = b2.reshape(1, D)

    eint = jnp.stack([ei[0].reshape(NCH, EPC), ei[1].reshape(NCH, EPC)],
                     axis=1)
    degw = _sc_degree(eint, zrows, ones_rows).reshape(NC, NP_, D)
    acc1 = _sc_aggregate(x, eint, zrows).reshape(NC, NP_, D)
    h = _tc_combine(acc1, degw, x, W_l1, W_r1, b1r, True)
    acc2 = _sc_aggregate(h, eint, zrows).reshape(NC, NP_, D)
    out = _tc_combine(acc2, degw, h, W_l2, W_r2, b2r, False)
    return out
